# baseline (speedup 1.0000x reference)
import functools

import jax
import jax.numpy as jnp
from jax import lax
from jax.experimental import pallas as pl
from jax.experimental.pallas import tpu as pltpu
from jax.experimental.pallas import tpu_sc as plsc

VOCAB = 100000
EMB = 128
CS = 64
CTX = 50
NEG = 10

NROW = CTX + CTX * NEG + 1
NC = 2
NS = 16
NW = NC * NS
CHUNK = 24
NBUF = 6
B_PAD = CHUNK * NW
E_ROWS = 64
E_CHUNK = 8

@functools.cache
def _sc_gather_fn():
    mesh = plsc.VectorSubcoreMesh(core_axis_name="c", subcore_axis_name="s",
                                  num_cores=NC, num_subcores=NS)

    @functools.partial(
        pl.kernel,
        mesh=mesh,
        compiler_params=pltpu.CompilerParams(use_tc_tiling_on_sc=True,
                                             needs_layout_passes=False),
        out_type=[
            jax.ShapeDtypeStruct((B_PAD, CS), jnp.float32),
            jax.ShapeDtypeStruct((B_PAD, CS), jnp.float32),
            jax.ShapeDtypeStruct((E_ROWS, EMB), jnp.float32),
        ],
        scratch_types=[
            pltpu.VMEM((CHUNK,), jnp.int32),
            pltpu.VMEM((NBUF, CS, 128), jnp.float32),
            pltpu.VMEM((NBUF, CS, 128), jnp.float32),
            pltpu.VMEM((CHUNK, CS), jnp.float32),
            pltpu.VMEM((CHUNK, CS), jnp.float32),
            pltpu.VMEM((E_CHUNK,), jnp.int32),
            pltpu.VMEM((E_CHUNK, EMB), jnp.float32),
            pltpu.SemaphoreType.DMA,
            pltpu.SemaphoreType.DMA,
            pltpu.SemaphoreType.DMA,
        ],
    )
    def _sc_gather(idx_hbm, eidx_hbm, mus_t_hbm, sigs_t_hbm, e_hbm,
                   out_mus, out_sigs, out_e,
                   idx_v, mblk, sblk, mstage, sstage, eidx_v, erows,
                   sem_m, sem_s, sem_e):
        wid = lax.axis_index("s") * NC + lax.axis_index("c")
        base = wid * CHUNK
        pltpu.sync_copy(idx_hbm.at[pl.ds(base, CHUNK)], idx_v)
        vecs = [idx_v[pl.ds(0, 16)], idx_v[pl.ds(8, 16)]]

        def scalar_idx(j):
            return vecs[0][j] if j < 16 else vecs[1][j - 8]

        def start(j):
            i = scalar_idx(j)
            c = lax.rem(i, 128)
            i0 = pl.multiple_of(i - c, 128)
            b = j % NBUF
            cm = pltpu.async_copy(mus_t_hbm.at[:, pl.ds(i0, 128)],
                                  mblk.at[b], sem_m)
            cs = pltpu.async_copy(sigs_t_hbm.at[:, pl.ds(i0, 128)],
                                  sblk.at[b], sem_s)
            return c, cm, cs

        rows0 = lax.iota(jnp.int32, 16)
        pend = [start(j) for j in range(NBUF)]

        @pl.when(wid < E_ROWS // E_CHUNK)
        def _():
            ebase = wid * E_CHUNK
            pltpu.sync_copy(eidx_hbm.at[pl.ds(ebase, E_CHUNK)], eidx_v)
            pltpu.async_copy(e_hbm.at[eidx_v], erows, sem_e)

        for j in range(CHUNK):
            c, cm, cs = pend[j % NBUF]
            cm.wait()
            cs.wait()
            b = j % NBUF
            cols = jnp.full((16,), c, jnp.int32)
            for t in range(CS // 16):
                rows = rows0 + (16 * t)
                mstage[j, pl.ds(16 * t, 16)] = plsc.load_gather(
                    mblk.at[b], [rows, cols])
                sstage[j, pl.ds(16 * t, 16)] = plsc.load_gather(
                    sblk.at[b], [rows, cols])
            if j + NBUF < CHUNK:
                pend[(j + NBUF) % NBUF] = start(j + NBUF)

        pltpu.sync_copy(mstage, out_mus.at[pl.ds(base, CHUNK)])
        pltpu.sync_copy(sstage, out_sigs.at[pl.ds(base, CHUNK)])

        @pl.when(wid < E_ROWS // E_CHUNK)
        def _():
            ebase = wid * E_CHUNK
            pltpu.make_async_copy(e_hbm.at[eidx_v], erows, sem_e).wait()
            pltpu.sync_copy(erows, out_e.at[pl.ds(ebase, E_CHUNK)])

    return _sc_gather


def _tc_body(gmu_ref, gsg_ref, ge_ref, mw_ref, mb_ref, uw_ref,
             ub_ref, ww_ref, wb_ref, out_ref):
    ge = ge_ref[...]
    r = jnp.dot(ge, mw_ref[...].T, preferred_element_type=jnp.float32)
    r = jnp.maximum(r + mb_ref[...], 0.0)
    h1 = r[0:1] * float(CTX)
    h2 = jnp.sum(r[1:CTX + 1], axis=0, keepdims=True)
    h = jnp.concatenate([h1, h2], axis=1)
    mu = jnp.dot(h, uw_ref[...].T, preferred_element_type=jnp.float32) + ub_ref[...]
    z = jnp.dot(h, ww_ref[...].T, preferred_element_type=jnp.float32) + wb_ref[...]
    post_var = jnp.maximum(z, 0.0) + jnp.log1p(jnp.exp(-jnp.abs(z)))
    lpv = jnp.sum(jnp.log(post_var))

    gmu = gmu_ref[...]
    gsg = gsg_ref[...]
    v = gsg * gsg
    d = gmu - mu
    t = jnp.sum((post_var + d * d) / v + jnp.log(v), axis=1, keepdims=True)
    kl = 0.5 * (t - (float(CS) + lpv))
    kl_pos = kl[0:CTX]
    kl_neg = kl[CTX:CTX + CTX * NEG].reshape(CTX, NEG)
    hinge = jnp.maximum(kl_neg - kl_pos + 1.0, 0.0)
    res = jnp.sum(hinge) - kl[CTX + CTX * NEG, 0]
    out_ref[...] = jnp.broadcast_to(res, (1, 1))


def kernel(x, context, neg_samples, E, M_w, M_b, U_w, U_b, W_w, W_b,
           prior_mus, prior_sigmas):
    x = x.astype(jnp.int32)
    context = context.astype(jnp.int32)
    neg = neg_samples.astype(jnp.int32)
    idx_all = jnp.concatenate(
        [context, neg.reshape(-1), x,
         jnp.zeros((B_PAD - NROW,), jnp.int32)])
    idx_e = jnp.concatenate(
        [x, context, jnp.zeros((E_ROWS - CTX - 1,), jnp.int32)])
    gmu, gsg, ge = _sc_gather_fn()(idx_all, idx_e, prior_mus.T,
                                   prior_sigmas.T, E)

    out = pl.pallas_call(
        _tc_body,
        out_shape=jax.ShapeDtypeStruct((1, 1), jnp.float32),
    )(gmu, gsg, ge, M_w, M_b.reshape(1, CS), U_w, U_b.reshape(1, CS),
      W_w, W_b.reshape(1, CS))
    return out.reshape(1)


# speedup vs baseline: 1.0164x; 1.0164x over previous
import functools

import jax
import jax.numpy as jnp
from jax import lax
from jax.experimental import pallas as pl
from jax.experimental.pallas import tpu as pltpu
from jax.experimental.pallas import tpu_sc as plsc

VOCAB = 100000
EMB = 128
CS = 64
CTX = 50
NEG = 10

NROW = CTX + CTX * NEG + 1
NC = 2
NS = 16
NW = NC * NS
CHUNK = 24
NBUF = 6
B_PAD = CHUNK * NW
E_ROWS = 64
E_CHUNK = 8
E_LAST_BASE = 544
E_X_ROW = 56 + (CTX + CTX * NEG) - E_LAST_BASE

@functools.cache
def _sc_gather_fn():
    mesh = plsc.VectorSubcoreMesh(core_axis_name="c", subcore_axis_name="s",
                                  num_cores=NC, num_subcores=NS)

    @functools.partial(
        pl.kernel,
        mesh=mesh,
        compiler_params=pltpu.CompilerParams(use_tc_tiling_on_sc=True,
                                             needs_layout_passes=False),
        out_type=[
            jax.ShapeDtypeStruct((B_PAD, CS), jnp.float32),
            jax.ShapeDtypeStruct((B_PAD, CS), jnp.float32),
            jax.ShapeDtypeStruct((E_ROWS, EMB), jnp.float32),
        ],
        scratch_types=[
            pltpu.VMEM((CHUNK,), jnp.int32),
            pltpu.VMEM((NBUF, CS, 128), jnp.float32),
            pltpu.VMEM((NBUF, CS, 128), jnp.float32),
            pltpu.VMEM((CHUNK, CS), jnp.float32),
            pltpu.VMEM((CHUNK, CS), jnp.float32),
            pltpu.VMEM((E_CHUNK,), jnp.int32),
            pltpu.VMEM((E_CHUNK, EMB), jnp.float32),
            pltpu.SemaphoreType.DMA,
            pltpu.SemaphoreType.DMA,
            pltpu.SemaphoreType.DMA,
        ],
    )
    def _sc_gather(idx_hbm, mus_t_hbm, sigs_t_hbm, e_hbm,
                   out_mus, out_sigs, out_e,
                   idx_v, mblk, sblk, mstage, sstage, eidx_v, erows,
                   sem_m, sem_s, sem_e):
        wid = lax.axis_index("s") * NC + lax.axis_index("c")
        base = wid * CHUNK
        pltpu.sync_copy(idx_hbm.at[pl.ds(base, CHUNK)], idx_v)
        vecs = [idx_v[pl.ds(0, 16)], idx_v[pl.ds(8, 16)]]

        def scalar_idx(j):
            return vecs[0][j] if j < 16 else vecs[1][j - 8]

        def start(j):
            i = scalar_idx(j)
            c = lax.rem(i, 128)
            i0 = pl.multiple_of(i - c, 128)
            b = j % NBUF
            cm = pltpu.async_copy(mus_t_hbm.at[:, pl.ds(i0, 128)],
                                  mblk.at[b], sem_m)
            cs = pltpu.async_copy(sigs_t_hbm.at[:, pl.ds(i0, 128)],
                                  sblk.at[b], sem_s)
            return c, cm, cs

        rows0 = lax.iota(jnp.int32, 16)
        pend = [start(j) for j in range(NBUF)]

        @pl.when(wid < E_ROWS // E_CHUNK)
        def _():
            ebase = lax.select(wid == E_ROWS // E_CHUNK - 1,
                               jnp.int32(E_LAST_BASE),
                               (wid * E_CHUNK).astype(jnp.int32))
            pltpu.sync_copy(idx_hbm.at[pl.ds(ebase, E_CHUNK)], eidx_v)
            pltpu.async_copy(e_hbm.at[eidx_v], erows, sem_e)

        for j in range(CHUNK):
            c, cm, cs = pend[j % NBUF]
            cm.wait()
            cs.wait()
            b = j % NBUF
            cols = jnp.full((16,), c, jnp.int32)
            for t in range(CS // 16):
                rows = rows0 + (16 * t)
                mstage[j, pl.ds(16 * t, 16)] = plsc.load_gather(
                    mblk.at[b], [rows, cols])
                sstage[j, pl.ds(16 * t, 16)] = plsc.load_gather(
                    sblk.at[b], [rows, cols])
            if j + NBUF < CHUNK:
                pend[(j + NBUF) % NBUF] = start(j + NBUF)

        pltpu.sync_copy(mstage, out_mus.at[pl.ds(base, CHUNK)])
        pltpu.sync_copy(sstage, out_sigs.at[pl.ds(base, CHUNK)])

        @pl.when(wid < E_ROWS // E_CHUNK)
        def _():
            ebase = wid * E_CHUNK
            pltpu.make_async_copy(e_hbm.at[eidx_v], erows, sem_e).wait()
            pltpu.sync_copy(erows, out_e.at[pl.ds(ebase, E_CHUNK)])

    return _sc_gather


def _tc_body(gmu_ref, gsg_ref, ge_ref, mw_ref, mb_ref, uw_ref,
             ub_ref, ww_ref, wb_ref, out_ref):
    ge = ge_ref[...]
    r = jnp.dot(ge, mw_ref[...].T, preferred_element_type=jnp.float32)
    r = jnp.maximum(r + mb_ref[...], 0.0)
    h1 = r[E_X_ROW:E_X_ROW + 1] * float(CTX)
    h2 = jnp.sum(r[0:CTX], axis=0, keepdims=True)
    h = jnp.concatenate([h1, h2], axis=1)
    mu = jnp.dot(h, uw_ref[...].T, preferred_element_type=jnp.float32) + ub_ref[...]
    z = jnp.dot(h, ww_ref[...].T, preferred_element_type=jnp.float32) + wb_ref[...]
    post_var = jnp.maximum(z, 0.0) + jnp.log1p(jnp.exp(-jnp.abs(z)))
    lpv = jnp.sum(jnp.log(post_var))

    gmu = gmu_ref[...]
    gsg = gsg_ref[...]
    v = gsg * gsg
    d = gmu - mu
    t = jnp.sum((post_var + d * d) / v + jnp.log(v), axis=1, keepdims=True)
    kl = 0.5 * (t - (float(CS) + lpv))
    kl_pos = kl[0:CTX]
    kl_neg = kl[CTX:CTX + CTX * NEG].reshape(CTX, NEG)
    hinge = jnp.maximum(kl_neg - kl_pos + 1.0, 0.0)
    res = jnp.sum(hinge) - kl[CTX + CTX * NEG, 0]
    out_ref[...] = jnp.broadcast_to(res, (1, 1))


def kernel(x, context, neg_samples, E, M_w, M_b, U_w, U_b, W_w, W_b,
           prior_mus, prior_sigmas):
    x = x.astype(jnp.int32)
    context = context.astype(jnp.int32)
    neg = neg_samples.astype(jnp.int32)
    idx_all = jnp.concatenate(
        [context, neg.reshape(-1), x,
         jnp.zeros((B_PAD - NROW,), jnp.int32)])
    gmu, gsg, ge = _sc_gather_fn()(idx_all, prior_mus.T, prior_sigmas.T, E)

    out = pl.pallas_call(
        _tc_body,
        out_shape=jax.ShapeDtypeStruct((1, 1), jnp.float32),
    )(gmu, gsg, ge, M_w, M_b.reshape(1, CS), U_w, U_b.reshape(1, CS),
      W_w, W_b.reshape(1, CS))
    return out.reshape(1)


# speedup vs baseline: 1.5751x; 1.5496x over previous
import functools

import jax
import jax.numpy as jnp
from jax import lax
from jax.experimental import pallas as pl
from jax.experimental.pallas import tpu as pltpu
from jax.experimental.pallas import tpu_sc as plsc

VOCAB = 100000
EMB = 128
CS = 64
CTX = 50
NEG = 10

NROW = CTX + CTX * NEG + 1
NC = 2
NS = 16
NW = NC * NS
CHUNK = 16
NBUF = 4
B_SC = CHUNK * NW
B_TC = 64
IDX_LEN = B_SC + B_TC
E_ROWS = 64
E_CHUNK = 8
E_LAST_BASE = 544
E_X_ROW = 56 + (CTX + CTX * NEG) - E_LAST_BASE


@functools.cache
def _sc_gather_fn():
    mesh = plsc.VectorSubcoreMesh(core_axis_name="c", subcore_axis_name="s",
                                  num_cores=NC, num_subcores=NS)

    @functools.partial(
        pl.kernel,
        mesh=mesh,
        compiler_params=pltpu.CompilerParams(use_tc_tiling_on_sc=True,
                                             needs_layout_passes=False),
        out_type=[
            jax.ShapeDtypeStruct((B_SC, CS), jnp.float32),
            jax.ShapeDtypeStruct((B_SC, CS), jnp.float32),
            jax.ShapeDtypeStruct((E_ROWS, EMB), jnp.float32),
        ],
        scratch_types=[
            pltpu.VMEM((CHUNK,), jnp.int32),
            pltpu.VMEM((NBUF, CS, 128), jnp.float32),
            pltpu.VMEM((NBUF, CS, 128), jnp.float32),
            pltpu.VMEM((CHUNK, CS), jnp.float32),
            pltpu.VMEM((CHUNK, CS), jnp.float32),
            pltpu.VMEM((E_CHUNK,), jnp.int32),
            pltpu.VMEM((E_CHUNK, EMB), jnp.float32),
            pltpu.SemaphoreType.DMA,
            pltpu.SemaphoreType.DMA,
            pltpu.SemaphoreType.DMA,
        ],
    )
    def _sc_gather(idx_hbm, mus_t_hbm, sigs_t_hbm, e_hbm,
                   out_mus, out_sigs, out_e,
                   idx_v, mblk, sblk, mstage, sstage, eidx_v, erows,
                   sem_m, sem_s, sem_e):
        wid = lax.axis_index("s") * NC + lax.axis_index("c")
        base = wid * CHUNK
        pltpu.sync_copy(idx_hbm.at[pl.ds(base, CHUNK)], idx_v)
        vec = idx_v[pl.ds(0, 16)]

        def start(j):
            i = vec[j]
            c = lax.rem(i, 128)
            i0 = pl.multiple_of(i - c, 128)
            b = j % NBUF
            cm = pltpu.async_copy(mus_t_hbm.at[:, pl.ds(i0, 128)],
                                  mblk.at[b], sem_m)
            cs = pltpu.async_copy(sigs_t_hbm.at[:, pl.ds(i0, 128)],
                                  sblk.at[b], sem_s)
            return c, cm, cs

        rows0 = lax.iota(jnp.int32, 16)
        pend = [start(j) for j in range(NBUF)]

        @pl.when(wid < E_ROWS // E_CHUNK)
        def _():
            ebase = lax.select(wid == E_ROWS // E_CHUNK - 1,
                               jnp.int32(E_LAST_BASE),
                               (wid * E_CHUNK).astype(jnp.int32))
            pltpu.sync_copy(idx_hbm.at[pl.ds(ebase, E_CHUNK)], eidx_v)
            pltpu.async_copy(e_hbm.at[eidx_v], erows, sem_e)

        for j in range(CHUNK):
            c, cm, cs = pend[j % NBUF]
            cm.wait()
            cs.wait()
            b = j % NBUF
            cols = jnp.full((16,), c, jnp.int32)
            for t in range(CS // 16):
                rows = rows0 + (16 * t)
                mstage[j, pl.ds(16 * t, 16)] = plsc.load_gather(
                    mblk.at[b], [rows, cols])
                sstage[j, pl.ds(16 * t, 16)] = plsc.load_gather(
                    sblk.at[b], [rows, cols])
            if j + NBUF < CHUNK:
                pend[(j + NBUF) % NBUF] = start(j + NBUF)

        pltpu.sync_copy(mstage, out_mus.at[pl.ds(base, CHUNK)])
        pltpu.sync_copy(sstage, out_sigs.at[pl.ds(base, CHUNK)])

        @pl.when(wid < E_ROWS // E_CHUNK)
        def _():
            ebase = wid * E_CHUNK
            pltpu.make_async_copy(e_hbm.at[eidx_v], erows, sem_e).wait()
            pltpu.sync_copy(erows, out_e.at[pl.ds(ebase, E_CHUNK)])

    return _sc_gather


def _tc_gather_body(idx_ref, mus_t_ref, sigs_t_ref, out_m_ref, out_s_ref,
                    mblk, sblk, sem_m, sem_s):
    copies = []
    cvals = []
    for j in range(B_TC):
        i = idx_ref[j]
        c = lax.rem(i, 128)
        i0 = pl.multiple_of(i - c, 128)
        copies.append(pltpu.make_async_copy(
            mus_t_ref.at[:, pl.ds(i0, 128)], mblk.at[j], sem_m))
        copies.append(pltpu.make_async_copy(
            sigs_t_ref.at[:, pl.ds(i0, 128)], sblk.at[j], sem_s))
        copies[-2].start()
        copies[-1].start()
        cvals.append(c)
    lanes = lax.broadcasted_iota(jnp.int32, (1, 128), 1)
    for j in range(B_TC):
        copies[2 * j].wait()
        copies[2 * j + 1].wait()
        onehot = (lanes == cvals[j]).astype(jnp.float32)
        out_m_ref[:, pl.ds(j, 1)] = jnp.sum(mblk[j] * onehot, axis=1,
                                            keepdims=True)
        out_s_ref[:, pl.ds(j, 1)] = jnp.sum(sblk[j] * onehot, axis=1,
                                            keepdims=True)


def _tc_body(gmu_ref, gsg_ref, gmu2_ref, gsg2_ref, ge_ref, mw_ref, mb_ref,
             uw_ref, ub_ref, ww_ref, wb_ref, out_ref):
    ge = ge_ref[...]
    r = jnp.dot(ge, mw_ref[...].T, preferred_element_type=jnp.float32)
    r = jnp.maximum(r + mb_ref[...], 0.0)
    h1 = r[E_X_ROW:E_X_ROW + 1] * float(CTX)
    h2 = jnp.sum(r[0:CTX], axis=0, keepdims=True)
    h = jnp.concatenate([h1, h2], axis=1)
    mu = jnp.dot(h, uw_ref[...].T, preferred_element_type=jnp.float32) + ub_ref[...]
    z = jnp.dot(h, ww_ref[...].T, preferred_element_type=jnp.float32) + wb_ref[...]
    post_var = jnp.maximum(z, 0.0) + jnp.log1p(jnp.exp(-jnp.abs(z)))
    lpv = jnp.sum(jnp.log(post_var))

    gmu = gmu_ref[...]
    gsg = gsg_ref[...]
    v = gsg * gsg
    d = gmu - mu
    t1 = jnp.sum((post_var + d * d) / v + jnp.log(v), axis=1, keepdims=True)

    gmu2 = gmu2_ref[...]
    gsg2 = gsg2_ref[...]
    mu_c = mu.reshape(CS, 1)
    pv_c = post_var.reshape(CS, 1)
    v2 = gsg2 * gsg2
    d2 = gmu2 - mu_c
    term2 = (pv_c + d2 * d2) / v2 + jnp.log(v2)
    t2 = jnp.dot(term2.T, jnp.ones((CS, 1), jnp.float32),
                 preferred_element_type=jnp.float32)

    kl = 0.5 * (jnp.concatenate([t1, t2], axis=0)
                - (float(CS) + lpv))
    kl_pos = kl[0:CTX]
    kl_neg = kl[CTX:CTX + CTX * NEG].reshape(CTX, NEG)
    hinge = jnp.maximum(kl_neg - kl_pos + 1.0, 0.0)
    res = jnp.sum(hinge) - kl[CTX + CTX * NEG, 0]
    out_ref[...] = jnp.broadcast_to(res, (1, 1))


def kernel(x, context, neg_samples, E, M_w, M_b, U_w, U_b, W_w, W_b,
           prior_mus, prior_sigmas):
    x = x.astype(jnp.int32)
    context = context.astype(jnp.int32)
    neg = neg_samples.astype(jnp.int32)
    idx_all = jnp.concatenate(
        [context, neg.reshape(-1), x,
         jnp.zeros((IDX_LEN - NROW,), jnp.int32)])
    mus_t = prior_mus.T
    sigs_t = prior_sigmas.T
    gmu, gsg, ge = _sc_gather_fn()(idx_all, mus_t, sigs_t, E)

    gmu2, gsg2 = pl.pallas_call(
        _tc_gather_body,
        in_specs=[
            pl.BlockSpec(memory_space=pltpu.SMEM),
            pl.BlockSpec(memory_space=pl.ANY),
            pl.BlockSpec(memory_space=pl.ANY),
        ],
        out_shape=[
            jax.ShapeDtypeStruct((CS, B_TC), jnp.float32),
            jax.ShapeDtypeStruct((CS, B_TC), jnp.float32),
        ],
        scratch_shapes=[
            pltpu.VMEM((B_TC, CS, 128), jnp.float32),
            pltpu.VMEM((B_TC, CS, 128), jnp.float32),
            pltpu.SemaphoreType.DMA,
            pltpu.SemaphoreType.DMA,
        ],
    )(idx_all[B_SC:IDX_LEN], mus_t, sigs_t)

    out = pl.pallas_call(
        _tc_body,
        out_shape=jax.ShapeDtypeStruct((1, 1), jnp.float32),
    )(gmu, gsg, gmu2, gsg2, ge, M_w, M_b.reshape(1, CS), U_w,
      U_b.reshape(1, CS), W_w, W_b.reshape(1, CS))
    return out.reshape(1)
